# baseline (speedup 1.0000x reference)
import functools

import jax
import jax.numpy as jnp
from jax import lax
from jax.experimental import pallas as pl
from jax.experimental.pallas import tpu as pltpu
from jax.experimental.pallas import tpu_sc as plsc

_B, _C, _A, _W, _O, _L, _K = 1024, 100, 6, 5, 4, 256, 2
_NC, _NS = 2, 16
_NW = _NC * _NS
_IPW = (_B * _K) // _NW
_NEG = float("-inf")


def _sc_gather_rows(stab, etab, colf):
  mesh = plsc.VectorSubcoreMesh(core_axis_name="c", subcore_axis_name="s")
  f32 = jnp.float32

  @functools.partial(
      pl.kernel,
      mesh=mesh,
      out_type=[
          jax.ShapeDtypeStruct((_B * _K, _L), f32),
          jax.ShapeDtypeStruct((_B * _K, _L), f32),
      ],
      scratch_types=[
          pltpu.VMEM((_IPW,), jnp.int32),
          pltpu.VMEM((_IPW, _L), f32),
          pltpu.VMEM((_IPW, _L), f32),
          pltpu.SemaphoreType.DMA,
          pltpu.SemaphoreType.DMA,
      ],
      compiler_params=pltpu.CompilerParams(needs_layout_passes=False),
  )
  def k(stab_hbm, etab_hbm, colf_hbm, srows_o, erows_o, idx_v, sr_v, er_v,
        sem_s, sem_e):
    wid = lax.axis_index("s") * _NC + lax.axis_index("c")
    i0 = wid * _IPW
    iota = lax.iota(jnp.int32, 16)
    pltpu.sync_copy(colf_hbm.at[pl.ds(i0, _IPW)], idx_v)
    for j in range(_IPW // 16):
      col = idx_v[pl.ds(16 * j, 16)]
      b = (i0 + 16 * j + iota) & (_B - 1)
      idx_v[pl.ds(16 * j, 16)] = col * _B + b
    cp_s = pltpu.async_copy(stab_hbm.at[idx_v], sr_v, sem_s)
    cp_e = pltpu.async_copy(etab_hbm.at[idx_v], er_v, sem_e)
    cp_s.wait()
    cp_e.wait()
    pltpu.sync_copy(sr_v, srows_o.at[pl.ds(i0, _IPW)])
    pltpu.sync_copy(er_v, erows_o.at[pl.ds(i0, _IPW)])

  return k(stab, etab, colf)


def _softplus(x):
  return jnp.maximum(x, 0.0) + jnp.log1p(jnp.exp(-jnp.abs(x)))


def _tc_dense_body(sel_ref, num_ref, col_ref, agg_ref, op_ref, selt_ref,
                   numt_ref, aggt_ref, colt_ref, opt_ref, out_ref):
  sel = sel_ref[...]
  ci = lax.broadcasted_iota(jnp.int32, (_C, _B), 0)
  m = jnp.max(sel, axis=0, keepdims=True)
  lse = m + jnp.log(jnp.sum(jnp.exp(sel - m), axis=0, keepdims=True))
  picked = jnp.sum(jnp.where(ci == selt_ref[...], sel, 0.0), axis=0,
                   keepdims=True)
  loss = jnp.sum(lse - picked) * (1.0 / _B)
  amax = jnp.min(jnp.where(sel == m, ci, _C), axis=0, keepdims=True)

  agg = agg_ref[...]
  ci3 = lax.broadcasted_iota(jnp.int32, (_A, _C, _B), 1)
  arow = jnp.sum(jnp.where(ci3 == amax[None], agg, 0.0), axis=1)
  ai = lax.broadcasted_iota(jnp.int32, (_A, _B), 0)
  ma = jnp.max(arow, axis=0, keepdims=True)
  lsea = ma + jnp.log(jnp.sum(jnp.exp(arow - ma), axis=0, keepdims=True))
  pa = jnp.sum(jnp.where(ai == aggt_ref[...], arow, 0.0), axis=0,
               keepdims=True)
  loss = loss + jnp.sum(lsea - pa) * (1.0 / _B)

  num = num_ref[...]
  wi = lax.broadcasted_iota(jnp.int32, (_W, _B), 0)
  mn = jnp.max(num, axis=0, keepdims=True)
  lsen = mn + jnp.log(jnp.sum(jnp.exp(num - mn), axis=0, keepdims=True))
  pn = jnp.sum(jnp.where(wi == numt_ref[...], num, 0.0), axis=0,
               keepdims=True)
  loss = loss + jnp.sum(lsen - pn) * (1.0 / _B)

  colw = col_ref[...]
  t0 = colt_ref[0:1, :]
  t1 = colt_ref[1:2, :]
  h = (ci == t0) | (ci == t1)
  sp_pos = _softplus(colw)
  sp_neg = sp_pos - colw
  loss = loss + jnp.sum(jnp.where(h, 3.0 * sp_neg, sp_pos)) * (
      jnp.float32(_B) / _C)

  op = op_ref[...]
  ci3o = lax.broadcasted_iota(jnp.int32, (_C, _O, _B), 0)
  oi = lax.broadcasted_iota(jnp.int32, (_O, _B), 0)
  for kk in range(_K):
    ck = colt_ref[kk:kk + 1, :]
    orow = jnp.sum(jnp.where(ci3o == ck[:, None], op, 0.0), axis=0)
    mo = jnp.max(orow, axis=0, keepdims=True)
    lseo = mo + jnp.log(jnp.sum(jnp.exp(orow - mo), axis=0, keepdims=True))
    po = jnp.sum(jnp.where(oi == opt_ref[kk:kk + 1, :], orow, 0.0), axis=0,
                 keepdims=True)
    loss = loss + jnp.sum(lseo - po) * (1.0 / (_B * _K))

  out_ref[...] = jnp.reshape(loss, (1, 1))


def _tc_rows_body(srow_ref, erow_ref, stt_ref, ett_ref, part_ref, out_ref):
  loss = part_ref[0, 0]
  jl = lax.broadcasted_iota(jnp.int32, (_B * _K, _L), 1)
  for rows_ref, tgt_ref in ((srow_ref, stt_ref), (erow_ref, ett_ref)):
    x = rows_ref[...]
    mr = jnp.max(x, axis=1, keepdims=True)
    lser = mr + jnp.log(jnp.sum(jnp.exp(x - mr), axis=1, keepdims=True))
    pr = jnp.sum(jnp.where(jl == tgt_ref[...], x, 0.0), axis=1,
                 keepdims=True)
    loss = loss + jnp.sum(lser - pr) * (1.0 / (_B * _K))

  out_ref[...] = jnp.reshape(loss, (1, 1))


def kernel(agg_logits, sel_logits, where_num_logits, where_col_logits,
           where_op_logits, where_start_logits, where_end_logits,
           agg_target, sel_target, where_num_target, where_col_target,
           where_op_target, where_start_target, where_end_target):
  i32 = jnp.int32
  colt_t = where_col_target.astype(i32).T

  srows, erows = _sc_gather_rows(
      where_start_logits.transpose(1, 0, 2).reshape(_C * _B, _L),
      where_end_logits.transpose(1, 0, 2).reshape(_C * _B, _L),
      colt_t.reshape(-1))

  part = pl.pallas_call(
      _tc_dense_body,
      out_shape=jax.ShapeDtypeStruct((1, 1), jnp.float32),
  )(
      sel_logits.T,
      where_num_logits.T,
      where_col_logits.T,
      agg_logits.transpose(2, 1, 0),
      where_op_logits.transpose(1, 2, 0),
      sel_target.astype(i32).reshape(1, _B),
      where_num_target.astype(i32).reshape(1, _B),
      agg_target.astype(i32).reshape(1, _B),
      colt_t,
      where_op_target.astype(i32).T,
  )

  out = pl.pallas_call(
      _tc_rows_body,
      out_shape=jax.ShapeDtypeStruct((1, 1), jnp.float32),
  )(
      srows,
      erows,
      where_start_target.astype(i32).T.reshape(_B * _K, 1),
      where_end_target.astype(i32).T.reshape(_B * _K, 1),
      part,
  )
  return out[0, 0]


# speedup vs baseline: 1.0259x; 1.0259x over previous
import functools

import jax
import jax.numpy as jnp
from jax import lax
from jax.experimental import pallas as pl
from jax.experimental.pallas import tpu as pltpu
from jax.experimental.pallas import tpu_sc as plsc

_B, _C, _A, _W, _O, _L, _K = 1024, 100, 6, 5, 4, 256, 2
_NC, _NS = 2, 16
_NW = _NC * _NS
_IPW = (_B * _K) // _NW
_NEG = float("-inf")


def _sc_gather_rows(stab, etab, idx):
  mesh = plsc.VectorSubcoreMesh(core_axis_name="c", subcore_axis_name="s")
  f32 = jnp.float32

  @functools.partial(
      pl.kernel,
      mesh=mesh,
      out_type=[
          jax.ShapeDtypeStruct((_B * _K, _L), f32),
          jax.ShapeDtypeStruct((_B * _K, _L), f32),
      ],
      scratch_types=[
          pltpu.VMEM((_IPW,), jnp.int32),
          pltpu.VMEM((_IPW, _L), f32),
          pltpu.VMEM((_IPW, _L), f32),
          pltpu.SemaphoreType.DMA,
          pltpu.SemaphoreType.DMA,
      ],
      compiler_params=pltpu.CompilerParams(needs_layout_passes=False),
  )
  def k(stab_hbm, etab_hbm, idx_hbm, srows_o, erows_o, idx_v, sr_v, er_v,
        sem_s, sem_e):
    wid = lax.axis_index("s") * _NC + lax.axis_index("c")
    i0 = wid * _IPW
    pltpu.sync_copy(idx_hbm.at[pl.ds(i0, _IPW)], idx_v)
    cp_s = pltpu.async_copy(stab_hbm.at[idx_v], sr_v, sem_s)
    cp_e = pltpu.async_copy(etab_hbm.at[idx_v], er_v, sem_e)
    cp_s.wait()
    cp_e.wait()
    pltpu.sync_copy(sr_v, srows_o.at[pl.ds(i0, _IPW)])
    pltpu.sync_copy(er_v, erows_o.at[pl.ds(i0, _IPW)])

  return k(stab, etab, idx)


def _softplus(x):
  return jnp.maximum(x, 0.0) + jnp.log1p(jnp.exp(-jnp.abs(x)))


def _tc_loss_body(sel_ref, num_ref, col_ref, agg_ref, op_ref, srow_ref,
                  erow_ref, selt_ref, numt_ref, aggt_ref, colt_ref, opt_ref,
                  stt_ref, ett_ref, out_ref):
  sel = sel_ref[...]
  ci = lax.broadcasted_iota(jnp.int32, (_C, _B), 0)
  m = jnp.max(sel, axis=0, keepdims=True)
  lse = m + jnp.log(jnp.sum(jnp.exp(sel - m), axis=0, keepdims=True))
  picked = jnp.sum(jnp.where(ci == selt_ref[...], sel, 0.0), axis=0,
                   keepdims=True)
  loss = jnp.sum(lse - picked) * (1.0 / _B)
  amax = jnp.min(jnp.where(sel == m, ci, _C), axis=0, keepdims=True)

  agg = agg_ref[...]
  ci3 = lax.broadcasted_iota(jnp.int32, (_A, _C, _B), 1)
  arow = jnp.sum(jnp.where(ci3 == amax[None], agg, 0.0), axis=1)
  ai = lax.broadcasted_iota(jnp.int32, (_A, _B), 0)
  ma = jnp.max(arow, axis=0, keepdims=True)
  lsea = ma + jnp.log(jnp.sum(jnp.exp(arow - ma), axis=0, keepdims=True))
  pa = jnp.sum(jnp.where(ai == aggt_ref[...], arow, 0.0), axis=0,
               keepdims=True)
  loss = loss + jnp.sum(lsea - pa) * (1.0 / _B)

  num = num_ref[...]
  wi = lax.broadcasted_iota(jnp.int32, (_W, _B), 0)
  mn = jnp.max(num, axis=0, keepdims=True)
  lsen = mn + jnp.log(jnp.sum(jnp.exp(num - mn), axis=0, keepdims=True))
  pn = jnp.sum(jnp.where(wi == numt_ref[...], num, 0.0), axis=0,
               keepdims=True)
  loss = loss + jnp.sum(lsen - pn) * (1.0 / _B)

  colw = col_ref[...]
  t0 = colt_ref[0:1, :]
  t1 = colt_ref[1:2, :]
  h = (ci == t0) | (ci == t1)
  sp_pos = _softplus(colw)
  sp_neg = sp_pos - colw
  loss = loss + jnp.sum(jnp.where(h, 3.0 * sp_neg, sp_pos)) * (
      jnp.float32(_B) / _C)

  op = op_ref[...]
  ci3o = lax.broadcasted_iota(jnp.int32, (_C, _O, _B), 0)
  oi = lax.broadcasted_iota(jnp.int32, (_O, _B), 0)
  for kk in range(_K):
    ck = colt_ref[kk:kk + 1, :]
    orow = jnp.sum(jnp.where(ci3o == ck[:, None], op, 0.0), axis=0)
    mo = jnp.max(orow, axis=0, keepdims=True)
    lseo = mo + jnp.log(jnp.sum(jnp.exp(orow - mo), axis=0, keepdims=True))
    po = jnp.sum(jnp.where(oi == opt_ref[kk:kk + 1, :], orow, 0.0), axis=0,
                 keepdims=True)
    loss = loss + jnp.sum(lseo - po) * (1.0 / (_B * _K))

  jl = lax.broadcasted_iota(jnp.int32, (_B * _K, _L), 1)
  for rows_ref, tgt_ref in ((srow_ref, stt_ref), (erow_ref, ett_ref)):
    x = rows_ref[...]
    mr = jnp.max(x, axis=1, keepdims=True)
    lser = mr + jnp.log(jnp.sum(jnp.exp(x - mr), axis=1, keepdims=True))
    pr = jnp.sum(jnp.where(jl == tgt_ref[...], x, 0.0), axis=1,
                 keepdims=True)
    loss = loss + jnp.sum(lser - pr) * (1.0 / (_B * _K))

  out_ref[...] = jnp.reshape(loss, (1, 1))


def kernel(agg_logits, sel_logits, where_num_logits, where_col_logits,
           where_op_logits, where_start_logits, where_end_logits,
           agg_target, sel_target, where_num_target, where_col_target,
           where_op_target, where_start_target, where_end_target):
  i32 = jnp.int32
  colt_t = where_col_target.astype(i32).T
  idx = (colt_t * _B + jnp.arange(_B, dtype=i32)[None, :]).reshape(-1)

  srows, erows = _sc_gather_rows(
      where_start_logits.transpose(1, 0, 2).reshape(_C * _B, _L),
      where_end_logits.transpose(1, 0, 2).reshape(_C * _B, _L),
      idx)

  out = pl.pallas_call(
      _tc_loss_body,
      out_shape=jax.ShapeDtypeStruct((1, 1), jnp.float32),
  )(
      sel_logits.T,
      where_num_logits.T,
      where_col_logits.T,
      agg_logits.transpose(2, 1, 0),
      where_op_logits.transpose(1, 2, 0),
      srows,
      erows,
      sel_target.astype(i32).reshape(1, _B),
      where_num_target.astype(i32).reshape(1, _B),
      agg_target.astype(i32).reshape(1, _B),
      colt_t,
      where_op_target.astype(i32).T,
      where_start_target.astype(i32).T.reshape(_B * _K, 1),
      where_end_target.astype(i32).T.reshape(_B * _K, 1),
  )
  return out[0, 0]
